# Initial kernel scaffold; baseline (speedup 1.0000x reference)
import jax
import jax.numpy as jnp
from jax.experimental import pallas as pl


def kernel(x, batch, size, fc_w1, fc_b1, fc_w2, fc_b2, weight_matrix):
    raise NotImplementedError("write your pallas kernel here")


# speedup vs baseline: 1.3616x; 1.3616x over previous
import jax
import jax.numpy as jnp
from jax.experimental import pallas as pl
from jax.experimental.pallas import tpu as pltpu

N = 160000
D = 256
S = 10000
R = 64

ROW_BLK = 2000


def _mlp_body(x_ref, w1_ref, b1_ref, w2_ref, b2_ref, out_ref):
    x = x_ref[...]
    h = jnp.maximum(jnp.dot(x, w1_ref[...], preferred_element_type=jnp.float32) + b1_ref[...], 0.0)
    a = jnp.tanh(jnp.dot(h, w2_ref[...], preferred_element_type=jnp.float32) + b2_ref[...])
    out_ref[...] = (a + 1.0) * x


def _x2(x, fc_w1, fc_b1, fc_w2, fc_b2):
    grid = N // ROW_BLK
    return pl.pallas_call(
        _mlp_body,
        grid=(grid,),
        in_specs=[
            pl.BlockSpec((ROW_BLK, D), lambda i: (i, 0)),
            pl.BlockSpec((D, R), lambda i: (0, 0)),
            pl.BlockSpec((R,), lambda i: (0,)),
            pl.BlockSpec((R, D), lambda i: (0, 0)),
            pl.BlockSpec((D,), lambda i: (0,)),
        ],
        out_specs=pl.BlockSpec((ROW_BLK, D), lambda i: (i, 0)),
        out_shape=jax.ShapeDtypeStruct((N, D), jnp.float32),
    )(x, fc_w1, fc_b1, fc_w2, fc_b2)


def kernel(x, batch, size, fc_w1, fc_b1, fc_w2, fc_b2, weight_matrix):
    x2 = _x2(x, fc_w1, fc_b1, fc_w2, fc_b2)
    seg_sum = jax.ops.segment_sum(x2, batch, num_segments=S)
    counts = jax.ops.segment_sum(jnp.ones((N,), jnp.float32), batch, num_segments=S)
    mean = seg_sum / jnp.maximum(counts, 1.0)[:, None]
    tg = jnp.tanh(mean @ weight_matrix)
    coefs = jax.nn.sigmoid(jnp.sum(x2 * tg[batch], axis=1))
    return jax.ops.segment_sum(coefs[:, None] * x2, batch, num_segments=S)
